# baseline (speedup 1.0000x reference)
import functools

import jax
import jax.numpy as jnp
from jax import lax
from jax.experimental import pallas as pl
from jax.experimental.pallas import tpu as pltpu
from jax.experimental.pallas import tpu_sc as plsc


def _sc_info():
    try:
        info = plsc.get_sparse_core_info()
        return info.num_cores, info.num_subcores
    except Exception:
        return 2, 16


def _tc_table_transform(cat_table, sub_table, W, cat_d, sub_d, emb_d):
    n_c = cat_table.shape[0]
    n_s = sub_table.shape[0]

    def body(ct_ref, st_ref, wc_ref, ws_ref, oc_ref, os_ref):
        oc_ref[...] = jnp.dot(ct_ref[...], wc_ref[...],
                              preferred_element_type=jnp.float32)
        os_ref[...] = jnp.dot(st_ref[...], ws_ref[...],
                              preferred_element_type=jnp.float32)

    return pl.pallas_call(
        body,
        out_shape=[
            jax.ShapeDtypeStruct((n_c, emb_d), jnp.float32),
            jax.ShapeDtypeStruct((n_s, emb_d), jnp.float32),
        ],
    )(cat_table, sub_table, W[0:cat_d], W[cat_d:cat_d + sub_d])


def _make_sc_pool(flat, n_sc, T, tok_d, emb_d, nc, ns):
    nw = nc * ns
    G = 16
    ids_per_chunk = G * T
    per_w = flat // nw
    n_chunks = per_w // G
    per_w_tok = n_sc // nw
    n_chunks_tok = per_w_tok // G
    assert per_w * nw == flat and n_chunks * G == per_w
    assert per_w_tok * nw == n_sc and n_chunks_tok * G == per_w_tok

    mesh = plsc.VectorSubcoreMesh(core_axis_name="c", subcore_axis_name="s")

    @functools.partial(
        pl.kernel,
        out_type=[
            jax.ShapeDtypeStruct((max(n_sc, 1), tok_d), jnp.float32),
            jax.ShapeDtypeStruct((flat, emb_d), jnp.float32),
        ],
        mesh=mesh,
        scratch_types=[
            pltpu.VMEM((ids_per_chunk,), jnp.int32),
            pltpu.VMEM((ids_per_chunk,), jnp.int32),
            pltpu.VMEM((ids_per_chunk, tok_d), jnp.float32),
            pltpu.VMEM((G, tok_d), jnp.float32),
            pltpu.VMEM((G,), jnp.int32),
            pltpu.VMEM((G,), jnp.int32),
            pltpu.VMEM((G, emb_d), jnp.float32),
            pltpu.VMEM((G, emb_d), jnp.float32),
            pltpu.SemaphoreType.DMA,
            pltpu.SemaphoreType.DMA,
            pltpu.SemaphoreType.DMA,
        ],
    )
    def pool_kernel(ids2_hbm, msk2_hbm, cat_ids_hbm, sub_ids_hbm,
                    tok_tab, cat_w, sub_w,
                    ts_out, cs_out,
                    ids_v, msk_v, rows_v, acc_v, cidx_v, sidx_v,
                    catrow_v, subrow_v, gsem, csem, ssem):
        wid = lax.axis_index("s") * nc + lax.axis_index("c")
        base_cs = wid * per_w
        base_tok = wid * per_w_tok

        def cs_body(g, carry):
            ib = base_cs + g * G
            pltpu.sync_copy(cat_ids_hbm.at[pl.ds(ib, G)], cidx_v)
            pltpu.sync_copy(sub_ids_hbm.at[pl.ds(ib, G)], sidx_v)
            pltpu.async_copy(cat_w.at[cidx_v], catrow_v, csem).wait()
            pltpu.async_copy(sub_w.at[sidx_v], subrow_v, ssem).wait()
            for i in range(G):
                for d in range(emb_d // 16):
                    sl = pl.ds(d * 16, 16)
                    catrow_v[i, sl] = catrow_v[i, sl] + subrow_v[i, sl]
            pltpu.sync_copy(catrow_v, cs_out.at[pl.ds(ib, G)])
            return carry

        lax.fori_loop(0, n_chunks, cs_body, 0)

        def tok_body(g, carry):
            ib = base_tok + g * G
            pltpu.sync_copy(ids2_hbm.at[pl.ds(ib * T, ids_per_chunk)], ids_v)
            pltpu.sync_copy(msk2_hbm.at[pl.ds(ib * T, ids_per_chunk)], msk_v)
            handles = []
            for k in range(0, ids_per_chunk, 16):
                sl = pl.ds(k, 16)
                idx = ids_v[sl] * msk_v[sl]
                handles.append(pltpu.async_copy(
                    tok_tab.at[idx], rows_v.at[sl], gsem))
            for h in handles:
                h.wait()

            def item_body(i, carry2):
                r = i * T
                for d in range(tok_d // 16):
                    sl = pl.ds(d * 16, 16)
                    acc = rows_v[r, sl]
                    for t in range(1, T):
                        acc = acc + rows_v[r + t, sl]
                    acc_v[i, sl] = acc
                return carry2

            lax.fori_loop(0, G, item_body, 0)
            pltpu.sync_copy(acc_v, ts_out.at[pl.ds(ib, G)])
            return carry

        if n_chunks_tok > 0:
            lax.fori_loop(0, n_chunks_tok, tok_body, 0)

    return pool_kernel


def _make_tc_pool(n_tc, T, vocab, tok_d, blk_items):
    nb = n_tc // blk_items
    assert nb * blk_items == n_tc
    seg = blk_items * T

    def body(ids_ref, msk_ref, tab_ref, o_ref):
        def item(i, carry):
            r = i * T
            idx0 = ids_ref[0, 0, r] * msk_ref[0, 0, r]
            acc = tab_ref[pl.ds(idx0, 1), :]
            for t in range(1, T):
                idx = ids_ref[0, 0, r + t] * msk_ref[0, 0, r + t]
                acc = acc + tab_ref[pl.ds(idx, 1), :]
            o_ref[pl.ds(i, 1), :] = acc
            return carry

        lax.fori_loop(0, blk_items, item, 0)

    return pl.pallas_call(
        body,
        grid=(nb,),
        in_specs=[
            pl.BlockSpec((1, 1, seg), lambda i: (i, 0, 0),
                         memory_space=pltpu.SMEM),
            pl.BlockSpec((1, 1, seg), lambda i: (i, 0, 0),
                         memory_space=pltpu.SMEM),
            pl.BlockSpec((vocab, tok_d), lambda i: (0, 0)),
        ],
        out_specs=pl.BlockSpec((blk_items, tok_d), lambda i: (i, 0)),
        out_shape=jax.ShapeDtypeStruct((n_tc, tok_d), jnp.float32),
    )


def _tc_head(ts, cs, maskf, Wt, b, gamma, beta, tok_d, emb_d):
    flat, T = maskf.shape
    BLK = 256
    grid = (flat // BLK,)

    def body(ts_ref, cs_ref, mf_ref, w_ref, b_ref, g_ref, be_ref, o_ref):
        cnt = jnp.sum(mf_ref[...], axis=1, keepdims=True)
        inv = 1.0 / jnp.maximum(cnt, 1.0)
        title = ts_ref[...] * inv
        h = jnp.dot(title, w_ref[...], preferred_element_type=jnp.float32)
        h = h + cs_ref[...] + b_ref[...]
        mu = jnp.mean(h, axis=1, keepdims=True)
        var = jnp.mean(jnp.square(h - mu), axis=1, keepdims=True)
        h = (h - mu) * lax.rsqrt(var + 1e-5) * g_ref[...] + be_ref[...]
        o_ref[...] = jnp.maximum(h, 0.0)

    return pl.pallas_call(
        body,
        grid=grid,
        in_specs=[
            pl.BlockSpec((BLK, tok_d), lambda i: (i, 0)),
            pl.BlockSpec((BLK, emb_d), lambda i: (i, 0)),
            pl.BlockSpec((BLK, T), lambda i: (i, 0)),
            pl.BlockSpec((tok_d, emb_d), lambda i: (0, 0)),
            pl.BlockSpec((1, emb_d), lambda i: (0, 0)),
            pl.BlockSpec((1, emb_d), lambda i: (0, 0)),
            pl.BlockSpec((1, emb_d), lambda i: (0, 0)),
        ],
        out_specs=pl.BlockSpec((BLK, emb_d), lambda i: (i, 0)),
        out_shape=jax.ShapeDtypeStruct((flat, emb_d), jnp.float32),
    )(ts, cs, maskf, Wt, b.reshape(1, emb_d), gamma.reshape(1, emb_d),
      beta.reshape(1, emb_d))


_SC_ITEM_SHARE = 0.08
_TC_BLK_ITEMS = 128


def kernel(category_ids, subcategory_ids, title_token_ids, title_token_mask,
           cat_table, subcat_table, tok_table, W, b, gamma, beta):
    Bb, Ll = category_ids.shape
    T = title_token_ids.shape[-1]
    flat = Bb * Ll
    cat_d = cat_table.shape[1]
    sub_d = subcat_table.shape[1]
    tok_d = tok_table.shape[1]
    emb_d = W.shape[1]
    vocab = tok_table.shape[0]

    nc, ns = _sc_info()
    nw = nc * ns
    quant = nw * 16
    n_sc = int(flat * _SC_ITEM_SHARE) // quant * quant
    while (flat - n_sc) % _TC_BLK_ITEMS != 0:
        n_sc -= quant
    n_tc = flat - n_sc

    ids_flat = title_token_ids.reshape(flat * T).astype(jnp.int32)
    msk_flat = title_token_mask.reshape(flat * T).astype(jnp.int32)
    cat_ids = category_ids.reshape(flat).astype(jnp.int32)
    sub_ids = subcategory_ids.reshape(flat).astype(jnp.int32)
    maskf = title_token_mask.reshape(flat, T).astype(jnp.float32)

    cat_w, sub_w = _tc_table_transform(cat_table, subcat_table, W,
                                       cat_d, sub_d, emb_d)
    sc_pool = _make_sc_pool(flat, n_sc, T, tok_d, emb_d, nc, ns)
    ts_sc, cs = sc_pool(ids_flat, msk_flat, cat_ids, sub_ids,
                        tok_table, cat_w, sub_w)
    tc_pool = _make_tc_pool(n_tc, T, vocab, tok_d, _TC_BLK_ITEMS)
    seg = _TC_BLK_ITEMS * T
    ts_tc = tc_pool(
        lax.dynamic_slice_in_dim(ids_flat, n_sc * T, n_tc * T)
        .reshape(n_tc // _TC_BLK_ITEMS, 1, seg),
        lax.dynamic_slice_in_dim(msk_flat, n_sc * T, n_tc * T)
        .reshape(n_tc // _TC_BLK_ITEMS, 1, seg),
        tok_table)
    if n_sc > 0:
        ts = jnp.concatenate([ts_sc, ts_tc], axis=0)
    else:
        ts = ts_tc
    h = _tc_head(ts, cs, maskf, W[cat_d + sub_d:], b, gamma, beta,
                 tok_d, emb_d)
    return h.reshape(Bb, Ll, emb_d)


# speedup vs baseline: 1.0561x; 1.0561x over previous
import functools

import jax
import jax.numpy as jnp
from jax import lax
from jax.experimental import pallas as pl
from jax.experimental.pallas import tpu as pltpu
from jax.experimental.pallas import tpu_sc as plsc


def _sc_info():
    try:
        info = plsc.get_sparse_core_info()
        return info.num_cores, info.num_subcores
    except Exception:
        return 2, 16


def _tc_table_transform(cat_table, sub_table, W, cat_d, sub_d, emb_d):
    n_c = cat_table.shape[0]
    n_s = sub_table.shape[0]

    def body(ct_ref, st_ref, wc_ref, ws_ref, oc_ref, os_ref):
        oc_ref[...] = jnp.dot(ct_ref[...], wc_ref[...],
                              preferred_element_type=jnp.float32)
        os_ref[...] = jnp.dot(st_ref[...], ws_ref[...],
                              preferred_element_type=jnp.float32)

    return pl.pallas_call(
        body,
        out_shape=[
            jax.ShapeDtypeStruct((n_c, emb_d), jnp.float32),
            jax.ShapeDtypeStruct((n_s, emb_d), jnp.float32),
        ],
    )(cat_table, sub_table, W[0:cat_d], W[cat_d:cat_d + sub_d])


def _make_sc_pool(flat, n_sc, T, tok_d, emb_d, nc, ns):
    nw = nc * ns
    G = 16
    ids_per_chunk = G * T
    per_w = flat // nw
    n_chunks = per_w // G
    per_w_tok = n_sc // nw
    n_chunks_tok = per_w_tok // G
    assert per_w * nw == flat and n_chunks * G == per_w
    assert per_w_tok * nw == n_sc and n_chunks_tok * G == per_w_tok

    mesh = plsc.VectorSubcoreMesh(core_axis_name="c", subcore_axis_name="s")

    @functools.partial(
        pl.kernel,
        out_type=[
            jax.ShapeDtypeStruct((max(n_sc, 1), tok_d), jnp.float32),
            jax.ShapeDtypeStruct((flat, emb_d), jnp.float32),
        ],
        mesh=mesh,
        scratch_types=[
            pltpu.VMEM((ids_per_chunk,), jnp.int32),
            pltpu.VMEM((ids_per_chunk,), jnp.int32),
            pltpu.VMEM((ids_per_chunk, tok_d), jnp.float32),
            pltpu.VMEM((G, tok_d), jnp.float32),
            pltpu.VMEM((G,), jnp.int32),
            pltpu.VMEM((G,), jnp.int32),
            pltpu.VMEM((G, emb_d), jnp.float32),
            pltpu.VMEM((G, emb_d), jnp.float32),
            pltpu.SemaphoreType.DMA,
            pltpu.SemaphoreType.DMA,
            pltpu.SemaphoreType.DMA,
        ],
    )
    def pool_kernel(ids2_hbm, msk2_hbm, cat_ids_hbm, sub_ids_hbm,
                    tok_tab, cat_w, sub_w,
                    ts_out, cs_out,
                    ids_v, msk_v, rows_v, acc_v, cidx_v, sidx_v,
                    catrow_v, subrow_v, gsem, csem, ssem):
        wid = lax.axis_index("s") * nc + lax.axis_index("c")
        base_cs = wid * per_w
        base_tok = wid * per_w_tok

        def cs_body(g, carry):
            ib = base_cs + g * G
            pltpu.sync_copy(cat_ids_hbm.at[pl.ds(ib, G)], cidx_v)
            pltpu.sync_copy(sub_ids_hbm.at[pl.ds(ib, G)], sidx_v)
            pltpu.async_copy(cat_w.at[cidx_v], catrow_v, csem).wait()
            pltpu.async_copy(sub_w.at[sidx_v], subrow_v, ssem).wait()
            for i in range(G):
                for d in range(emb_d // 16):
                    sl = pl.ds(d * 16, 16)
                    catrow_v[i, sl] = catrow_v[i, sl] + subrow_v[i, sl]
            pltpu.sync_copy(catrow_v, cs_out.at[pl.ds(ib, G)])
            return carry

        lax.fori_loop(0, n_chunks, cs_body, 0)

        def tok_body(g, carry):
            ib = base_tok + g * G
            pltpu.sync_copy(ids2_hbm.at[pl.ds(ib * T, ids_per_chunk)], ids_v)
            pltpu.sync_copy(msk2_hbm.at[pl.ds(ib * T, ids_per_chunk)], msk_v)
            handles = []
            for k in range(0, ids_per_chunk, 16):
                sl = pl.ds(k, 16)
                idx = ids_v[sl] * msk_v[sl]
                handles.append(pltpu.async_copy(
                    tok_tab.at[idx], rows_v.at[sl], gsem))
            for h in handles:
                h.wait()

            def item_body(i, carry2):
                r = i * T
                for d in range(tok_d // 16):
                    sl = pl.ds(d * 16, 16)
                    acc = rows_v[r, sl]
                    for t in range(1, T):
                        acc = acc + rows_v[r + t, sl]
                    acc_v[i, sl] = acc
                return carry2

            lax.fori_loop(0, G, item_body, 0)
            pltpu.sync_copy(acc_v, ts_out.at[pl.ds(ib, G)])
            return carry

        if n_chunks_tok > 0:
            lax.fori_loop(0, n_chunks_tok, tok_body, 0)

    return pool_kernel


def _make_tc_pool(n_tc, blk_off, T, vocab, tok_d, blk_items):
    nb = n_tc // blk_items
    assert nb * blk_items == n_tc
    seg = blk_items * T

    def body(ids_ref, msk_ref, tab_ref, o_ref):
        def item(i, carry):
            r = i * T
            idx0 = ids_ref[0, 0, r] * msk_ref[0, 0, r]
            acc = tab_ref[pl.ds(idx0, 1), :]
            for t in range(1, T):
                idx = ids_ref[0, 0, r + t] * msk_ref[0, 0, r + t]
                acc = acc + tab_ref[pl.ds(idx, 1), :]
            o_ref[pl.ds(i, 1), :] = acc
            return carry

        lax.fori_loop(0, blk_items, item, 0)

    return pl.pallas_call(
        body,
        grid=(nb,),
        in_specs=[
            pl.BlockSpec((1, 1, seg), lambda i: (i + blk_off, 0, 0),
                         memory_space=pltpu.SMEM),
            pl.BlockSpec((1, 1, seg), lambda i: (i + blk_off, 0, 0),
                         memory_space=pltpu.SMEM),
            pl.BlockSpec((vocab, tok_d), lambda i: (0, 0)),
        ],
        out_specs=pl.BlockSpec((blk_items, tok_d), lambda i: (i, 0)),
        out_shape=jax.ShapeDtypeStruct((n_tc, tok_d), jnp.float32),
    )


def _tc_head(ts, cs, maskf, Wt, b, gamma, beta, tok_d, emb_d):
    flat, T = maskf.shape
    BLK = 256
    grid = (flat // BLK,)

    def body(ts_ref, cs_ref, mf_ref, w_ref, b_ref, g_ref, be_ref, o_ref):
        cnt = jnp.sum(mf_ref[...].astype(jnp.float32), axis=1, keepdims=True)
        inv = 1.0 / jnp.maximum(cnt, 1.0)
        title = ts_ref[...] * inv
        h = jnp.dot(title, w_ref[...], preferred_element_type=jnp.float32)
        h = h + cs_ref[...] + b_ref[...]
        mu = jnp.mean(h, axis=1, keepdims=True)
        var = jnp.mean(jnp.square(h - mu), axis=1, keepdims=True)
        h = (h - mu) * lax.rsqrt(var + 1e-5) * g_ref[...] + be_ref[...]
        o_ref[...] = jnp.maximum(h, 0.0)

    return pl.pallas_call(
        body,
        grid=grid,
        in_specs=[
            pl.BlockSpec((BLK, tok_d), lambda i: (i, 0)),
            pl.BlockSpec((BLK, emb_d), lambda i: (i, 0)),
            pl.BlockSpec((BLK, T), lambda i: (i, 0)),
            pl.BlockSpec((tok_d, emb_d), lambda i: (0, 0)),
            pl.BlockSpec((1, emb_d), lambda i: (0, 0)),
            pl.BlockSpec((1, emb_d), lambda i: (0, 0)),
            pl.BlockSpec((1, emb_d), lambda i: (0, 0)),
        ],
        out_specs=pl.BlockSpec((BLK, emb_d), lambda i: (i, 0)),
        out_shape=jax.ShapeDtypeStruct((flat, emb_d), jnp.float32),
    )(ts, cs, maskf, Wt, b.reshape(1, emb_d), gamma.reshape(1, emb_d),
      beta.reshape(1, emb_d))


_SC_ITEM_SHARE = 0.06
_TC_BLK_ITEMS = 128


def kernel(category_ids, subcategory_ids, title_token_ids, title_token_mask,
           cat_table, subcat_table, tok_table, W, b, gamma, beta):
    Bb, Ll = category_ids.shape
    T = title_token_ids.shape[-1]
    flat = Bb * Ll
    cat_d = cat_table.shape[1]
    sub_d = subcat_table.shape[1]
    tok_d = tok_table.shape[1]
    emb_d = W.shape[1]
    vocab = tok_table.shape[0]

    nc, ns = _sc_info()
    nw = nc * ns
    quant = nw * 16
    n_sc = int(flat * _SC_ITEM_SHARE) // quant * quant
    while (flat - n_sc) % _TC_BLK_ITEMS != 0:
        n_sc -= quant
    n_tc = flat - n_sc

    ids_flat = title_token_ids.reshape(flat * T).astype(jnp.int32)
    msk_flat = title_token_mask.reshape(flat * T).astype(jnp.int32)
    cat_ids = category_ids.reshape(flat).astype(jnp.int32)
    sub_ids = subcategory_ids.reshape(flat).astype(jnp.int32)
    mask2d = msk_flat.reshape(flat, T)

    cat_w, sub_w = _tc_table_transform(cat_table, subcat_table, W,
                                       cat_d, sub_d, emb_d)
    sc_pool = _make_sc_pool(flat, n_sc, T, tok_d, emb_d, nc, ns)
    ts_sc, cs = sc_pool(ids_flat, msk_flat, cat_ids, sub_ids,
                        tok_table, cat_w, sub_w)
    seg = _TC_BLK_ITEMS * T
    blk_off = n_sc // _TC_BLK_ITEMS
    tc_pool = _make_tc_pool(n_tc, blk_off, T, vocab, tok_d, _TC_BLK_ITEMS)
    ts_tc = tc_pool(ids_flat.reshape(flat // _TC_BLK_ITEMS, 1, seg),
                    msk_flat.reshape(flat // _TC_BLK_ITEMS, 1, seg),
                    tok_table)
    if n_sc > 0:
        ts = jnp.concatenate([ts_sc, ts_tc], axis=0)
    else:
        ts = ts_tc
    h = _tc_head(ts, cs, mask2d, W[cat_d + sub_d:], b, gamma, beta,
                 tok_d, emb_d)
    return h.reshape(Bb, Ll, emb_d)


# speedup vs baseline: 1.2246x; 1.1596x over previous
import functools

import jax
import jax.numpy as jnp
from jax import lax
from jax.experimental import pallas as pl
from jax.experimental.pallas import tpu as pltpu
from jax.experimental.pallas import tpu_sc as plsc


def _sc_info():
    try:
        info = plsc.get_sparse_core_info()
        return info.num_cores, info.num_subcores
    except Exception:
        return 2, 16


def _tc_table_transform(cat_table, sub_table, W, cat_d, sub_d, emb_d):
    n_c = cat_table.shape[0]
    n_s = sub_table.shape[0]

    def body(ct_ref, st_ref, wc_ref, ws_ref, oc_ref, os_ref):
        oc_ref[...] = jnp.dot(ct_ref[...], wc_ref[...],
                              preferred_element_type=jnp.float32)
        os_ref[...] = jnp.dot(st_ref[...], ws_ref[...],
                              preferred_element_type=jnp.float32)

    return pl.pallas_call(
        body,
        out_shape=[
            jax.ShapeDtypeStruct((n_c, emb_d), jnp.float32),
            jax.ShapeDtypeStruct((n_s, emb_d), jnp.float32),
        ],
    )(cat_table, sub_table, W[0:cat_d], W[cat_d:cat_d + sub_d])


def _make_sc_pool(flat, n_sc, T, tok_d, emb_d, nc, ns):
    nw = nc * ns
    G = 16
    ids_per_chunk = G * T
    per_w = flat // nw
    n_chunks = per_w // G
    per_w_tok = n_sc // nw
    n_chunks_tok = per_w_tok // G
    assert per_w * nw == flat and n_chunks * G == per_w
    assert per_w_tok * nw == n_sc and n_chunks_tok * G == per_w_tok

    mesh = plsc.VectorSubcoreMesh(core_axis_name="c", subcore_axis_name="s")

    @functools.partial(
        pl.kernel,
        out_type=[
            jax.ShapeDtypeStruct((max(n_sc, 1), tok_d), jnp.float32),
            jax.ShapeDtypeStruct((flat, emb_d), jnp.float32),
        ],
        mesh=mesh,
        scratch_types=[
            pltpu.VMEM((ids_per_chunk,), jnp.int32),
            pltpu.VMEM((ids_per_chunk,), jnp.int32),
            pltpu.VMEM((ids_per_chunk, tok_d), jnp.float32),
            pltpu.VMEM((G, tok_d), jnp.float32),
            pltpu.VMEM((G,), jnp.int32),
            pltpu.VMEM((G,), jnp.int32),
            pltpu.VMEM((G, emb_d), jnp.float32),
            pltpu.VMEM((G, emb_d), jnp.float32),
            pltpu.SemaphoreType.DMA,
            pltpu.SemaphoreType.DMA,
            pltpu.SemaphoreType.DMA,
        ],
    )
    def pool_kernel(ids2_hbm, msk2_hbm, cat_ids_hbm, sub_ids_hbm,
                    tok_tab, cat_w, sub_w,
                    ts_out, cs_out,
                    ids_v, msk_v, rows_v, acc_v, cidx_v, sidx_v,
                    catrow_v, subrow_v, gsem, csem, ssem):
        wid = lax.axis_index("s") * nc + lax.axis_index("c")
        base_cs = wid * per_w
        base_tok = wid * per_w_tok

        def cs_body(g, carry):
            ib = base_cs + g * G
            pltpu.sync_copy(cat_ids_hbm.at[pl.ds(ib, G)], cidx_v)
            pltpu.sync_copy(sub_ids_hbm.at[pl.ds(ib, G)], sidx_v)
            pltpu.async_copy(cat_w.at[cidx_v], catrow_v, csem).wait()
            pltpu.async_copy(sub_w.at[sidx_v], subrow_v, ssem).wait()
            for i in range(G):
                for d in range(emb_d // 16):
                    sl = pl.ds(d * 16, 16)
                    catrow_v[i, sl] = catrow_v[i, sl] + subrow_v[i, sl]
            pltpu.sync_copy(catrow_v, cs_out.at[pl.ds(ib, G)])
            return carry

        lax.fori_loop(0, n_chunks, cs_body, 0)

        def tok_body(g, carry):
            ib = base_tok + g * G
            pltpu.sync_copy(ids2_hbm.at[pl.ds(ib * T, ids_per_chunk)], ids_v)
            pltpu.sync_copy(msk2_hbm.at[pl.ds(ib * T, ids_per_chunk)], msk_v)
            handles = []
            for k in range(0, ids_per_chunk, 16):
                sl = pl.ds(k, 16)
                idx = ids_v[sl] * msk_v[sl]
                handles.append(pltpu.async_copy(
                    tok_tab.at[idx], rows_v.at[sl], gsem))
            for h in handles:
                h.wait()

            def item_body(i, carry2):
                r = i * T
                for d in range(tok_d // 16):
                    sl = pl.ds(d * 16, 16)
                    acc = rows_v[r, sl]
                    for t in range(1, T):
                        acc = acc + rows_v[r + t, sl]
                    acc_v[i, sl] = acc
                return carry2

            lax.fori_loop(0, G, item_body, 0)
            pltpu.sync_copy(acc_v, ts_out.at[pl.ds(ib, G)])
            return carry

        if n_chunks_tok > 0:
            lax.fori_loop(0, n_chunks_tok, tok_body, 0)

    return pool_kernel


def _make_tc_pool(n_tc, blk_off, T, vocab, tok_d, blk_items):
    nb = n_tc // blk_items
    assert nb * blk_items == n_tc
    seg = blk_items * T

    def body(ids_ref, tab_ref, o_ref):
        def item(i, carry):
            r = i * T
            rows = [tab_ref[pl.ds(ids_ref[0, 0, r + t], 1), :]
                    for t in range(T)]
            while len(rows) > 1:
                rows = [rows[k] + rows[k + 1] if k + 1 < len(rows)
                        else rows[k] for k in range(0, len(rows), 2)]
            o_ref[pl.ds(i, 1), :] = rows[0]
            return carry

        lax.fori_loop(0, blk_items, item, 0)

    return pl.pallas_call(
        body,
        grid=(nb,),
        in_specs=[
            pl.BlockSpec((1, 1, seg), lambda i: (i + blk_off, 0, 0),
                         memory_space=pltpu.SMEM),
            pl.BlockSpec((vocab, tok_d), lambda i: (0, 0)),
        ],
        out_specs=pl.BlockSpec((blk_items, tok_d), lambda i: (i, 0)),
        out_shape=jax.ShapeDtypeStruct((n_tc, tok_d), jnp.float32),
    )


def _tc_head(ts, cs, maskf, Wt, b, gamma, beta, tok_d, emb_d):
    flat, T = maskf.shape
    BLK = 256
    grid = (flat // BLK,)

    def body(ts_ref, cs_ref, mf_ref, w_ref, b_ref, g_ref, be_ref, o_ref):
        cnt = jnp.sum(mf_ref[...].astype(jnp.float32), axis=1, keepdims=True)
        inv = 1.0 / jnp.maximum(cnt, 1.0)
        title = ts_ref[...] * inv
        h = jnp.dot(title, w_ref[...], preferred_element_type=jnp.float32)
        h = h + cs_ref[...] + b_ref[...]
        mu = jnp.mean(h, axis=1, keepdims=True)
        var = jnp.mean(jnp.square(h - mu), axis=1, keepdims=True)
        h = (h - mu) * lax.rsqrt(var + 1e-5) * g_ref[...] + be_ref[...]
        o_ref[...] = jnp.maximum(h, 0.0)

    return pl.pallas_call(
        body,
        grid=grid,
        in_specs=[
            pl.BlockSpec((BLK, tok_d), lambda i: (i, 0)),
            pl.BlockSpec((BLK, emb_d), lambda i: (i, 0)),
            pl.BlockSpec((BLK, T), lambda i: (i, 0)),
            pl.BlockSpec((tok_d, emb_d), lambda i: (0, 0)),
            pl.BlockSpec((1, emb_d), lambda i: (0, 0)),
            pl.BlockSpec((1, emb_d), lambda i: (0, 0)),
            pl.BlockSpec((1, emb_d), lambda i: (0, 0)),
        ],
        out_specs=pl.BlockSpec((BLK, emb_d), lambda i: (i, 0)),
        out_shape=jax.ShapeDtypeStruct((flat, emb_d), jnp.float32),
    )(ts, cs, maskf, Wt, b.reshape(1, emb_d), gamma.reshape(1, emb_d),
      beta.reshape(1, emb_d))


_SC_ITEM_SHARE = 0.06
_TC_BLK_ITEMS = 128


def kernel(category_ids, subcategory_ids, title_token_ids, title_token_mask,
           cat_table, subcat_table, tok_table, W, b, gamma, beta):
    Bb, Ll = category_ids.shape
    T = title_token_ids.shape[-1]
    flat = Bb * Ll
    cat_d = cat_table.shape[1]
    sub_d = subcat_table.shape[1]
    tok_d = tok_table.shape[1]
    emb_d = W.shape[1]
    vocab = tok_table.shape[0]

    nc, ns = _sc_info()
    nw = nc * ns
    quant = nw * 16
    n_sc = int(flat * _SC_ITEM_SHARE) // quant * quant
    while (flat - n_sc) % _TC_BLK_ITEMS != 0:
        n_sc -= quant
    n_tc = flat - n_sc

    ids_flat = title_token_ids.reshape(flat * T).astype(jnp.int32)
    msk_flat = title_token_mask.reshape(flat * T).astype(jnp.int32)
    cat_ids = category_ids.reshape(flat).astype(jnp.int32)
    sub_ids = subcategory_ids.reshape(flat).astype(jnp.int32)
    mask2d = msk_flat.reshape(flat, T)

    cat_w, sub_w = _tc_table_transform(cat_table, subcat_table, W,
                                       cat_d, sub_d, emb_d)
    sc_pool = _make_sc_pool(flat, n_sc, T, tok_d, emb_d, nc, ns)
    ts_sc, cs = sc_pool(ids_flat, msk_flat, cat_ids, sub_ids,
                        tok_table, cat_w, sub_w)
    seg = _TC_BLK_ITEMS * T
    blk_off = n_sc // _TC_BLK_ITEMS
    tc_pool = _make_tc_pool(n_tc, blk_off, T, vocab, tok_d, _TC_BLK_ITEMS)
    ids_m = ids_flat * msk_flat
    ts_tc = tc_pool(ids_m.reshape(flat // _TC_BLK_ITEMS, 1, seg),
                    tok_table)
    if n_sc > 0:
        ts = jnp.concatenate([ts_sc, ts_tc], axis=0)
    else:
        ts = ts_tc
    h = _tc_head(ts, cs, mask2d, W[cat_d + sub_d:], b, gamma, beta,
                 tok_d, emb_d)
    return h.reshape(Bb, Ll, emb_d)
